# baseline (speedup 1.0000x reference)
import functools

import jax
import jax.numpy as jnp
from jax import lax
from jax.experimental import pallas as pl
from jax.experimental.pallas import tpu as pltpu
from jax.experimental.pallas import tpu_sc as plsc

B = 16384
TIME_DIM = 16
NODE_DIM = 32
HID = 128

NC, NS = 2, 16
NW = NC * NS
BPW = B // NW
CHUNK = 128
NCH = BPW // CHUNK

BLK = 2048


def _gather_sc(tidx, nidx, time_table, node_table):
    mesh = plsc.VectorSubcoreMesh(core_axis_name="c", subcore_axis_name="s")

    @functools.partial(
        pl.kernel,
        mesh=mesh,
        compiler_params=pltpu.CompilerParams(use_tc_tiling_on_sc=False),
        out_type=[
            jax.ShapeDtypeStruct((B, TIME_DIM), jnp.float32),
            jax.ShapeDtypeStruct((B, TIME_DIM), jnp.float32),
            jax.ShapeDtypeStruct((B, TIME_DIM), jnp.float32),
        ],
        scratch_types=[
            pltpu.VMEM((BPW,), jnp.int32),
            pltpu.VMEM((BPW,), jnp.int32),
            pltpu.VMEM((BPW, TIME_DIM), jnp.float32),
            pltpu.VMEM((BPW, TIME_DIM), jnp.float32),
            pltpu.VMEM((BPW, TIME_DIM), jnp.float32),
            pltpu.SemaphoreType.DMA,
        ],
    )
    def gather_kernel(tidx_hbm, nidx_hbm, ttab_hbm, ntaba_hbm, ntabb_hbm,
                      temb_hbm, nemba_hbm, nembb_hbm,
                      tidx_v, nidx_v, trows_v, narows_v, nbrows_v, sem):
        wid = lax.axis_index("s") * NC + lax.axis_index("c")
        base = wid * BPW
        pltpu.sync_copy(tidx_hbm.at[pl.ds(base, BPW)], tidx_v)
        pltpu.sync_copy(nidx_hbm.at[pl.ds(base, BPW)], nidx_v)
        copies = []
        for j in range(NCH):
            s = j * CHUNK
            copies.append(pltpu.async_copy(
                ttab_hbm.at[tidx_v.at[pl.ds(s, CHUNK)]],
                trows_v.at[pl.ds(s, CHUNK)], sem))
            copies.append(pltpu.async_copy(
                ntaba_hbm.at[nidx_v.at[pl.ds(s, CHUNK)]],
                narows_v.at[pl.ds(s, CHUNK)], sem))
            copies.append(pltpu.async_copy(
                ntabb_hbm.at[nidx_v.at[pl.ds(s, CHUNK)]],
                nbrows_v.at[pl.ds(s, CHUNK)], sem))
        for c in copies:
            c.wait()
        pltpu.sync_copy(trows_v, temb_hbm.at[pl.ds(base, BPW)])
        pltpu.sync_copy(narows_v, nemba_hbm.at[pl.ds(base, BPW)])
        pltpu.sync_copy(nbrows_v, nembb_hbm.at[pl.ds(base, BPW)])

    return gather_kernel(tidx, nidx, time_table,
                         node_table[:, :TIME_DIM],
                         node_table[:, TIME_DIM:])


def _mlp_body(te_ref, na_ref, nb_ref, w1t_ref, w1a_ref, w1b_ref,
              b1_ref, w2_ref, b2_ref, out_ref):
    h = jnp.dot(te_ref[...], w1t_ref[...], preferred_element_type=jnp.float32)
    h = h + jnp.dot(na_ref[...], w1a_ref[...],
                    preferred_element_type=jnp.float32)
    h = h + jnp.dot(nb_ref[...], w1b_ref[...],
                    preferred_element_type=jnp.float32)
    h = jnp.maximum(h + b1_ref[...], 0.0)
    out_ref[...] = (
        jnp.dot(h, w2_ref[...], preferred_element_type=jnp.float32)
        + b2_ref[...])


def _mlp_tc(temb, nemba, nembb, W1, b1, W2, b2, interpret=False):
    w1t = W1[:TIME_DIM]
    w1a = W1[TIME_DIM:2 * TIME_DIM]
    w1b = W1[2 * TIME_DIM:]
    b1r = b1.reshape(1, HID)
    b2r = b2.reshape(1, 2)
    return pl.pallas_call(
        _mlp_body,
        grid=(B // BLK,),
        in_specs=[
            pl.BlockSpec((BLK, TIME_DIM), lambda i: (i, 0)),
            pl.BlockSpec((BLK, TIME_DIM), lambda i: (i, 0)),
            pl.BlockSpec((BLK, TIME_DIM), lambda i: (i, 0)),
            pl.BlockSpec((TIME_DIM, HID), lambda i: (0, 0)),
            pl.BlockSpec((TIME_DIM, HID), lambda i: (0, 0)),
            pl.BlockSpec((TIME_DIM, HID), lambda i: (0, 0)),
            pl.BlockSpec((1, HID), lambda i: (0, 0)),
            pl.BlockSpec((HID, 2), lambda i: (0, 0)),
            pl.BlockSpec((1, 2), lambda i: (0, 0)),
        ],
        out_specs=pl.BlockSpec((BLK, 2), lambda i: (i, 0)),
        out_shape=jax.ShapeDtypeStruct((B, 2), jnp.float32),
        interpret=interpret,
    )(temb, nemba, nembb, w1t, w1a, w1b, b1r, W2, b2r)


def kernel(time_bucket_idx, node_idx, node_table, time_table, W1, b1, W2, b2):
    temb, nemba, nembb = _gather_sc(time_bucket_idx, node_idx,
                                    time_table, node_table)
    return _mlp_tc(temb, nemba, nembb, W1, b1, W2, b2)


# speedup vs baseline: 2.3448x; 2.3448x over previous
import functools

import jax
import jax.numpy as jnp
from jax import lax
from jax.experimental import pallas as pl
from jax.experimental.pallas import tpu as pltpu
from jax.experimental.pallas import tpu_sc as plsc

B = 16384
TIME_DIM = 16
NODE_DIM = 32
HID = 128

NC, NS = 2, 16
NW = NC * NS
BPW = B // NW
CHUNK = 128
NCH = BPW // CHUNK

BLK = 2048


def _gather_sc(tidx, nidx, time_table, node_table):
    mesh = plsc.VectorSubcoreMesh(core_axis_name="c", subcore_axis_name="s")

    @functools.partial(
        pl.kernel,
        mesh=mesh,
        compiler_params=pltpu.CompilerParams(use_tc_tiling_on_sc=False),
        out_type=[
            jax.ShapeDtypeStruct((B, TIME_DIM), jnp.float32),
            jax.ShapeDtypeStruct((B, NODE_DIM), jnp.float32),
        ],
        scratch_types=[
            pltpu.VMEM((BPW,), jnp.int32),
            pltpu.VMEM((BPW,), jnp.int32),
            pltpu.VMEM((BPW, TIME_DIM), jnp.float32),
            pltpu.VMEM((BPW, NODE_DIM), jnp.float32),
            pltpu.SemaphoreType.DMA,
        ],
    )
    def gather_kernel(tidx_hbm, nidx_hbm, ttab_hbm, ntab_hbm,
                      temb_hbm, nemb_hbm,
                      tidx_v, nidx_v, trows_v, nrows_v, sem):
        wid = lax.axis_index("s") * NC + lax.axis_index("c")
        base = wid * BPW
        pltpu.sync_copy(tidx_hbm.at[pl.ds(base, BPW)], tidx_v)
        pltpu.sync_copy(nidx_hbm.at[pl.ds(base, BPW)], nidx_v)
        copies = []
        for j in range(NCH):
            s = j * CHUNK
            copies.append(pltpu.async_copy(
                ttab_hbm.at[tidx_v.at[pl.ds(s, CHUNK)]],
                trows_v.at[pl.ds(s, CHUNK)], sem))
            copies.append(pltpu.async_copy(
                ntab_hbm.at[nidx_v.at[pl.ds(s, CHUNK)]],
                nrows_v.at[pl.ds(s, CHUNK)], sem))
        for c in copies:
            c.wait()
        pltpu.sync_copy(trows_v, temb_hbm.at[pl.ds(base, BPW)])
        pltpu.sync_copy(nrows_v, nemb_hbm.at[pl.ds(base, BPW)])

    return gather_kernel(tidx, nidx, time_table, node_table)


def _mlp_body(te_ref, ne_ref, w1t_ref, w1n_ref, b1_ref, w2_ref, b2_ref,
              out_ref):
    h = jnp.dot(te_ref[...], w1t_ref[...], preferred_element_type=jnp.float32)
    h = h + jnp.dot(ne_ref[...], w1n_ref[...],
                    preferred_element_type=jnp.float32)
    h = jnp.maximum(h + b1_ref[...], 0.0)
    out_ref[...] = (
        jnp.dot(h, w2_ref[...], preferred_element_type=jnp.float32)
        + b2_ref[...])


def _mlp_tc(temb, nemb, W1, b1, W2, b2, interpret=False):
    w1t = W1[:TIME_DIM]
    w1n = W1[TIME_DIM:]
    b1r = b1.reshape(1, HID)
    b2r = b2.reshape(1, 2)
    return pl.pallas_call(
        _mlp_body,
        grid=(B // BLK,),
        in_specs=[
            pl.BlockSpec((BLK, TIME_DIM), lambda i: (i, 0)),
            pl.BlockSpec((BLK, NODE_DIM), lambda i: (i, 0)),
            pl.BlockSpec((TIME_DIM, HID), lambda i: (0, 0)),
            pl.BlockSpec((NODE_DIM, HID), lambda i: (0, 0)),
            pl.BlockSpec((1, HID), lambda i: (0, 0)),
            pl.BlockSpec((HID, 2), lambda i: (0, 0)),
            pl.BlockSpec((1, 2), lambda i: (0, 0)),
        ],
        out_specs=pl.BlockSpec((BLK, 2), lambda i: (i, 0)),
        out_shape=jax.ShapeDtypeStruct((B, 2), jnp.float32),
        interpret=interpret,
    )(temb, nemb, w1t, w1n, b1r, W2, b2r)


def kernel(time_bucket_idx, node_idx, node_table, time_table, W1, b1, W2, b2):
    temb, nemb = _gather_sc(time_bucket_idx, node_idx, time_table, node_table)
    return _mlp_tc(temb, nemb, W1, b1, W2, b2)
